# baseline (speedup 1.0000x reference)
import functools

import jax
import jax.numpy as jnp
from jax import lax
from jax.experimental import pallas as pl
from jax.experimental.pallas import tpu as pltpu
from jax.experimental.pallas import tpu_sc as plsc

N = 10000
E = 160000
H = 1280
INF = 10
OUTF = 7

NR = 10240
PAD = N
EP = 163840
B = 128
NB32 = 40
BA = 128
NBQ = 20
NQ = 4
FC = 128
NCH = 10
RPT = NR // 16

BLK = 256
NBLK = NR // BLK
EPS = 1e-5
NF = float(N)

_mesh = plsc.VectorSubcoreMesh(core_axis_name="c", subcore_axis_name="s")


@functools.partial(
    pl.kernel,
    out_type=jax.ShapeDtypeStruct((2, NR, FC), jnp.float32),
    mesh=_mesh,
    scratch_types=[
        pltpu.VMEM((NB32, 1, B), jnp.int32),
        pltpu.VMEM((B, FC), jnp.float32),
        pltpu.VMEM_SHARED((NR, FC), jnp.float32),
    ],
)
def _deg_kernel(dst32, ones_hbm, zrows, out, dst_v, ones_v, deg_sh):
    c = lax.axis_index("c")
    s = lax.axis_index("s")
    wid = s * 2 + c
    pltpu.sync_copy(ones_hbm, ones_v)
    pltpu.sync_copy(dst32.at[wid], dst_v)
    pltpu.sync_copy(zrows, deg_sh.at[pl.ds(s * RPT, RPT)])
    plsc.subcore_barrier()

    def body(b, carry):
        pltpu.sync_copy(ones_v, deg_sh.at[dst_v.at[b, 0]], add=True)
        return carry

    lax.fori_loop(0, NB32, body, 0)
    plsc.subcore_barrier()
    pltpu.sync_copy(deg_sh.at[pl.ds(s * RPT, RPT)],
                    out.at[c, pl.ds(s * RPT, RPT)])


@functools.partial(
    pl.kernel,
    out_type=jax.ShapeDtypeStruct((NCH, NR, FC), jnp.float32),
    mesh=_mesh,
    scratch_types=[
        pltpu.VMEM((NBQ, 1, BA), jnp.int32),
        pltpu.VMEM((NBQ, 1, BA), jnp.int32),
        pltpu.VMEM((BA, FC), jnp.float32),
        pltpu.VMEM((BA, FC), jnp.float32),
        pltpu.VMEM_SHARED((NR, FC), jnp.float32),
        pltpu.SemaphoreType.DMA,
        pltpu.SemaphoreType.DMA,
        pltpu.SemaphoreType.DMA,
        pltpu.SemaphoreType.DMA,
    ],
)
def _agg_kernel(h3t, srcadj, dst16, zrows, out,
                src_v, dst_v, buf0, buf1,
                a_sh, gs0, gs1, ss0, ss1):
    c = lax.axis_index("c")
    s = lax.axis_index("s")
    bufs = (buf0, buf1)
    gsems = (gs0, gs1)
    ssems = (ss0, ss1)

    def issue_g(b, k):
        pltpu.async_copy(h3t.at[src_v.at[b, 0]], bufs[k], gsems[k])

    def drain_g(k):
        pltpu.make_async_copy(h3t.at[src_v.at[0, 0]], bufs[k],
                              gsems[k]).wait()

    def issue_s(b, k):
        pltpu.async_copy(bufs[k], a_sh.at[dst_v.at[b, 0]], ssems[k],
                         add=True)

    def drain_s(k):
        pltpu.make_async_copy(bufs[k], a_sh.at[dst_v.at[0, 0]],
                              ssems[k]).wait()

    for p in range(NCH // 2):
        ci = c * (NCH // 2) + p
        pltpu.sync_copy(zrows, a_sh.at[pl.ds(s * RPT, RPT)])
        plsc.subcore_barrier()

        for q in range(NQ):
            pltpu.sync_copy(srcadj.at[ci, s, q], src_v)
            pltpu.sync_copy(dst16.at[s, q], dst_v)
            issue_g(0, 0)

            def body(g, carry):
                for k in range(2):
                    b = g * 2 + k
                    k2 = (k + 1) % 2
                    drain_g(k)
                    issue_s(b, k)

                    @pl.when(b >= 1)
                    def _():
                        drain_s(k2)

                    @pl.when(b + 1 < NBQ)
                    def _():
                        issue_g(b + 1, k2)

                return carry

            lax.fori_loop(0, NBQ // 2, body, 0)
            drain_s((NBQ - 1) % 2)

        plsc.subcore_barrier()
        pltpu.sync_copy(a_sh.at[pl.ds(s * RPT, RPT)],
                        out.at[ci, pl.ds(s * RPT, RPT)])
        plsc.subcore_barrier()


def _stem_body(x_ref, w_ref, b_ref, d_ref, h_ref, dis_ref):
    h = jnp.dot(x_ref[...], w_ref[...], preferred_element_type=jnp.float32)
    h_ref[...] = jnp.maximum(h + b_ref[...], 0.0)
    deg = d_ref[0, :, 0:1] + d_ref[1, :, 0:1] + 1.0
    dis_ref[...] = lax.rsqrt(deg)


def _stem_call(x_pad, w, b2, degp):
    return pl.pallas_call(
        _stem_body,
        grid=(NBLK,),
        in_specs=[
            pl.BlockSpec((BLK, INF), lambda r: (r, 0)),
            pl.BlockSpec((INF, H), lambda r: (0, 0)),
            pl.BlockSpec((1, H), lambda r: (0, 0)),
            pl.BlockSpec((2, BLK, FC), lambda r: (0, r, 0)),
        ],
        out_specs=[
            pl.BlockSpec((BLK, H), lambda r: (r, 0)),
            pl.BlockSpec((BLK, 1), lambda r: (r, 0)),
        ],
        out_shape=[
            jax.ShapeDtypeStruct((NR, H), jnp.float32),
            jax.ShapeDtypeStruct((NR, 1), jnp.float32),
        ],
    )(x_pad, w, b2, degp)


def _bn_affine(sums_ref, g_ref, b_ref):
    mean = sums_ref[0:1, :] * (1.0 / NF)
    var = sums_ref[1:2, :] * (1.0 / NF) - mean * mean
    a = g_ref[...] * lax.rsqrt(var + EPS)
    cshift = b_ref[...] - mean * a
    return a, cshift


def _pre_body(h_ref, sums_ref, g_ref, b_ref, w_ref, dis_ref, out_ref):
    a, cshift = _bn_affine(sums_ref, g_ref, b_ref)
    hn = h_ref[...] * a + cshift
    res = jnp.dot(hn, w_ref[...], preferred_element_type=jnp.float32)
    res = res * dis_ref[...]
    for k in range(NCH):
        out_ref[k] = res[:, k * FC:(k + 1) * FC]


def _pre_call(hraw, sums, g2, b2, w, dis):
    return pl.pallas_call(
        _pre_body,
        grid=(NBLK,),
        in_specs=[
            pl.BlockSpec((BLK, H), lambda r: (r, 0)),
            pl.BlockSpec((2, H), lambda r: (0, 0)),
            pl.BlockSpec((1, H), lambda r: (0, 0)),
            pl.BlockSpec((1, H), lambda r: (0, 0)),
            pl.BlockSpec((H, H), lambda r: (0, 0)),
            pl.BlockSpec((BLK, 1), lambda r: (r, 0)),
        ],
        out_specs=pl.BlockSpec((NCH, BLK, FC), lambda r: (0, r, 0)),
        out_shape=jax.ShapeDtypeStruct((NCH, NR, FC), jnp.float32),
    )(hraw, sums, g2, b2, w, dis)


def _post_body(h_ref, sums_ref, g_ref, b_ref, agg_ref, h3_ref, dis_ref,
               gb_ref, lw_ref, lb_ref, out_ref, ns_ref):
    r = pl.program_id(0)
    a, cshift = _bn_affine(sums_ref, g_ref, b_ref)
    hprev = h_ref[...] * a + cshift
    agg = jnp.concatenate([agg_ref[k] + h3_ref[k] for k in range(NCH)],
                          axis=1)
    t = jnp.maximum(agg * dis_ref[...] + gb_ref[...], 0.0)
    u = jnp.dot(t, lw_ref[...], preferred_element_type=jnp.float32)
    u = jnp.maximum(u + lb_ref[...], 0.0)
    hnew = hprev + u
    out_ref[...] = hnew
    rows = r * BLK + lax.broadcasted_iota(jnp.int32, (BLK, 1), 0)
    msk = jnp.where(rows < N, 1.0, 0.0)
    hm = hnew * msk
    s_blk = jnp.sum(hm, axis=0, keepdims=True)
    ss_blk = jnp.sum(hnew * hm, axis=0, keepdims=True)
    blk_sums = jnp.concatenate([s_blk, ss_blk], axis=0)

    @pl.when(r == 0)
    def _():
        ns_ref[...] = blk_sums

    @pl.when(r > 0)
    def _():
        ns_ref[...] = ns_ref[...] + blk_sums


def _post_call(hraw, sums, g2, b2, aggc, h3c, dis, gb2, lw, lb2):
    return pl.pallas_call(
        _post_body,
        grid=(NBLK,),
        in_specs=[
            pl.BlockSpec((BLK, H), lambda r: (r, 0)),
            pl.BlockSpec((2, H), lambda r: (0, 0)),
            pl.BlockSpec((1, H), lambda r: (0, 0)),
            pl.BlockSpec((1, H), lambda r: (0, 0)),
            pl.BlockSpec((NCH, BLK, FC), lambda r: (0, r, 0)),
            pl.BlockSpec((NCH, BLK, FC), lambda r: (0, r, 0)),
            pl.BlockSpec((BLK, 1), lambda r: (r, 0)),
            pl.BlockSpec((1, H), lambda r: (0, 0)),
            pl.BlockSpec((H, H), lambda r: (0, 0)),
            pl.BlockSpec((1, H), lambda r: (0, 0)),
        ],
        out_specs=[
            pl.BlockSpec((BLK, H), lambda r: (r, 0)),
            pl.BlockSpec((2, H), lambda r: (0, 0)),
        ],
        out_shape=[
            jax.ShapeDtypeStruct((NR, H), jnp.float32),
            jax.ShapeDtypeStruct((2, H), jnp.float32),
        ],
    )(hraw, sums, g2, b2, aggc, h3c, dis, gb2, lw, lb2)


def _head_body(h_ref, sums_ref, g_ref, b_ref, w_ref, ob_ref, out_ref):
    a, cshift = _bn_affine(sums_ref, g_ref, b_ref)
    hn = h_ref[...] * a + cshift
    y = jnp.dot(hn, w_ref[...], preferred_element_type=jnp.float32)
    out_ref[...] = y + ob_ref[...]


def _head_call(hraw, sums, g2, b2, w, ob2):
    return pl.pallas_call(
        _head_body,
        grid=(NBLK,),
        in_specs=[
            pl.BlockSpec((BLK, H), lambda r: (r, 0)),
            pl.BlockSpec((2, H), lambda r: (0, 0)),
            pl.BlockSpec((1, H), lambda r: (0, 0)),
            pl.BlockSpec((1, H), lambda r: (0, 0)),
            pl.BlockSpec((H, OUTF), lambda r: (0, 0)),
            pl.BlockSpec((1, OUTF), lambda r: (0, 0)),
        ],
        out_specs=pl.BlockSpec((BLK, OUTF), lambda r: (r, 0)),
        out_shape=jax.ShapeDtypeStruct((NR, OUTF), jnp.float32),
    )(hraw, sums, g2, b2, w, ob2)


def kernel(x, params, edge_index):
    f32 = jnp.float32
    i32 = jnp.int32

    x_pad = jnp.zeros((NR, INF), f32).at[:N].set(x)
    pad_idx = jnp.full((EP - E,), PAD, dtype=i32)
    srcp = jnp.concatenate([edge_index[0].astype(i32), pad_idx])
    dstp = jnp.concatenate([edge_index[1].astype(i32), pad_idx])
    srcadj = (srcp[None, :]
              + (jnp.arange(NCH, dtype=i32) * NR)[:, None]
              ).reshape(NCH, 16, NQ, NBQ, 1, BA)
    dst16 = dstp.reshape(16, NQ, NBQ, 1, BA)
    dst32 = dstp.reshape(32, NB32, 1, B)
    ones_hbm = jnp.ones((B, FC), f32)
    zrows = jnp.zeros((RPT, FC), f32)

    def r2(v):
        return v.reshape(1, -1)

    degp = _deg_kernel(dst32, ones_hbm, zrows)
    hraw, dis = _stem_call(x_pad, params['stem_W'], r2(params['stem_b']), degp)

    sums = jnp.concatenate(
        [jnp.zeros((1, H), f32), jnp.full((1, H), NF * (1.0 - EPS), f32)],
        axis=0)
    g2 = jnp.ones((1, H), f32)
    b2 = jnp.zeros((1, H), f32)

    for i in range(5):
        h3c = _pre_call(hraw, sums, g2, b2, params['gcn_W%d' % i], dis)
        aggc = _agg_kernel(h3c.reshape(NCH * NR, FC), srcadj, dst16, zrows)
        hraw, sums = _post_call(
            hraw, sums, g2, b2, aggc, h3c, dis,
            r2(params['gcn_b%d' % i]), params['lin_W%d' % i],
            r2(params['lin_b%d' % i]))
        g2 = r2(params['bn_g%d' % i])
        b2 = r2(params['bn_b%d' % i])

    y = _head_call(hraw, sums, g2, b2, params['out_W'], r2(params['out_b']))
    return y[:N]


# speedup vs baseline: 1.0905x; 1.0905x over previous
import functools

import jax
import jax.numpy as jnp
from jax import lax
from jax.experimental import pallas as pl
from jax.experimental.pallas import tpu as pltpu
from jax.experimental.pallas import tpu_sc as plsc

N = 10000
E = 160000
H = 1280
INF = 10
OUTF = 7

NR = 10240
PAD = N
EP = 163840
B = 128
NB32 = 40
BA = 128
NBQ = 40
NQ = 2
DW = 128
FC = 128
NCH = 10
RPT = NR // 16

BLK = 256
NBLK = NR // BLK
EPS = 1e-5
NF = float(N)

_mesh = plsc.VectorSubcoreMesh(core_axis_name="c", subcore_axis_name="s")


@functools.partial(
    pl.kernel,
    out_type=jax.ShapeDtypeStruct((2, NR, DW), jnp.float32),
    mesh=_mesh,
    scratch_types=[
        pltpu.VMEM((NB32, 1, B), jnp.int32),
        pltpu.VMEM((B, DW), jnp.float32),
        pltpu.VMEM_SHARED((NR, DW), jnp.float32),
    ],
)
def _deg_kernel(dst32, ones_hbm, z16, out, dst_v, ones_v, deg_sh):
    c = lax.axis_index("c")
    s = lax.axis_index("s")
    wid = s * 2 + c
    pltpu.sync_copy(ones_hbm, ones_v)
    pltpu.sync_copy(dst32.at[wid], dst_v)
    pltpu.sync_copy(z16, deg_sh.at[pl.ds(s * RPT, RPT)])
    plsc.subcore_barrier()

    def body(b, carry):
        pltpu.sync_copy(ones_v, deg_sh.at[dst_v.at[b, 0]], add=True)
        return carry

    lax.fori_loop(0, NB32, body, 0)
    plsc.subcore_barrier()
    pltpu.sync_copy(deg_sh.at[pl.ds(s * RPT, RPT)],
                    out.at[c, pl.ds(s * RPT, RPT)])


@functools.partial(
    pl.kernel,
    out_type=jax.ShapeDtypeStruct((NCH, NR, FC), jnp.float32),
    mesh=_mesh,
    scratch_types=[
        pltpu.VMEM((NBQ, 1, BA), jnp.int32),
        pltpu.VMEM((NBQ, 1, BA), jnp.int32),
        pltpu.VMEM((BA, FC), jnp.float32),
        pltpu.VMEM((BA, FC), jnp.float32),
        pltpu.VMEM_SHARED((NR, FC), jnp.float32),
        pltpu.SemaphoreType.DMA,
        pltpu.SemaphoreType.DMA,
        pltpu.SemaphoreType.DMA,
        pltpu.SemaphoreType.DMA,
    ],
)
def _agg_kernel(h3t, srcadj, dst16, zrows, out,
                src_v, dst_v, buf0, buf1,
                a_sh, gs0, gs1, ss0, ss1):
    c = lax.axis_index("c")
    s = lax.axis_index("s")
    bufs = (buf0, buf1)
    gsems = (gs0, gs1)
    ssems = (ss0, ss1)

    def issue_g(b, k):
        pltpu.async_copy(h3t.at[src_v.at[b, 0]], bufs[k], gsems[k])

    def drain_g(k):
        pltpu.make_async_copy(h3t.at[src_v.at[0, 0]], bufs[k],
                              gsems[k]).wait()

    def issue_s(b, k):
        pltpu.async_copy(bufs[k], a_sh.at[dst_v.at[b, 0]], ssems[k],
                         add=True)

    def drain_s(k):
        pltpu.make_async_copy(bufs[k], a_sh.at[dst_v.at[0, 0]],
                              ssems[k]).wait()

    def prime(p, q):
        ci_ = c * (NCH // 2) + p
        pltpu.sync_copy(srcadj.at[ci_, s, q], src_v)
        pltpu.sync_copy(dst16.at[s, q], dst_v)
        issue_g(0, 0)
        issue_g(1, 1)

    prime(0, 0)
    for p in range(NCH // 2):
        ci = c * (NCH // 2) + p
        pltpu.sync_copy(zrows, a_sh.at[pl.ds(s * RPT, RPT)])
        plsc.subcore_barrier()

        for q in range(NQ):
            def body(g, carry):
                for k in range(2):
                    b = g * 2 + k
                    drain_g(k)
                    issue_s(b, k)
                    drain_s(k)

                    @pl.when(b + 2 < NBQ)
                    def _():
                        issue_g(b + 2, k)

                return carry

            lax.fori_loop(0, NBQ // 2, body, 0)
            if q + 1 < NQ:
                prime(p, q + 1)
            elif p + 1 < NCH // 2:
                prime(p + 1, 0)

        plsc.subcore_barrier()
        pltpu.sync_copy(a_sh.at[pl.ds(s * RPT, RPT)],
                        out.at[ci, pl.ds(s * RPT, RPT)])
        plsc.subcore_barrier()


def _stem_body(x_ref, w_ref, b_ref, d_ref, h_ref, dis_ref):
    h = jnp.dot(x_ref[...], w_ref[...], preferred_element_type=jnp.float32)
    h_ref[...] = jnp.maximum(h + b_ref[...], 0.0)
    deg = d_ref[0, :, 0:1] + d_ref[1, :, 0:1] + 1.0
    dis_ref[...] = lax.rsqrt(deg)


def _stem_call(x_pad, w, b2, degp):
    return pl.pallas_call(
        _stem_body,
        grid=(NBLK,),
        in_specs=[
            pl.BlockSpec((BLK, INF), lambda r: (r, 0)),
            pl.BlockSpec((INF, H), lambda r: (0, 0)),
            pl.BlockSpec((1, H), lambda r: (0, 0)),
            pl.BlockSpec((2, BLK, DW), lambda r: (0, r, 0)),
        ],
        out_specs=[
            pl.BlockSpec((BLK, H), lambda r: (r, 0)),
            pl.BlockSpec((BLK, 1), lambda r: (r, 0)),
        ],
        out_shape=[
            jax.ShapeDtypeStruct((NR, H), jnp.float32),
            jax.ShapeDtypeStruct((NR, 1), jnp.float32),
        ],
    )(x_pad, w, b2, degp)


def _bn_affine(sums_ref, g_ref, b_ref):
    mean = sums_ref[0:1, :] * (1.0 / NF)
    var = sums_ref[1:2, :] * (1.0 / NF) - mean * mean
    a = g_ref[...] * lax.rsqrt(var + EPS)
    cshift = b_ref[...] - mean * a
    return a, cshift


def _pre_body(h_ref, sums_ref, g_ref, b_ref, w_ref, dis_ref, out_ref):
    a, cshift = _bn_affine(sums_ref, g_ref, b_ref)
    hn = h_ref[...] * a + cshift
    res = jnp.dot(hn, w_ref[...], preferred_element_type=jnp.float32)
    res = res * dis_ref[...]
    for k in range(NCH):
        out_ref[k] = res[:, k * FC:(k + 1) * FC]


def _pre_call(hraw, sums, g2, b2, w, dis):
    return pl.pallas_call(
        _pre_body,
        grid=(NBLK,),
        in_specs=[
            pl.BlockSpec((BLK, H), lambda r: (r, 0)),
            pl.BlockSpec((2, H), lambda r: (0, 0)),
            pl.BlockSpec((1, H), lambda r: (0, 0)),
            pl.BlockSpec((1, H), lambda r: (0, 0)),
            pl.BlockSpec((H, H), lambda r: (0, 0)),
            pl.BlockSpec((BLK, 1), lambda r: (r, 0)),
        ],
        out_specs=pl.BlockSpec((NCH, BLK, FC), lambda r: (0, r, 0)),
        out_shape=jax.ShapeDtypeStruct((NCH, NR, FC), jnp.float32),
    )(hraw, sums, g2, b2, w, dis)


def _post_body(h_ref, sums_ref, g_ref, b_ref, agg_ref, h3_ref, dis_ref,
               gb_ref, lw_ref, lb_ref, out_ref, ns_ref):
    r = pl.program_id(0)
    a, cshift = _bn_affine(sums_ref, g_ref, b_ref)
    hprev = h_ref[...] * a + cshift
    agg = jnp.concatenate([agg_ref[k] + h3_ref[k] for k in range(NCH)],
                          axis=1)
    t = jnp.maximum(agg * dis_ref[...] + gb_ref[...], 0.0)
    u = jnp.dot(t, lw_ref[...], preferred_element_type=jnp.float32)
    u = jnp.maximum(u + lb_ref[...], 0.0)
    hnew = hprev + u
    out_ref[...] = hnew
    rows = r * BLK + lax.broadcasted_iota(jnp.int32, (BLK, 1), 0)
    msk = jnp.where(rows < N, 1.0, 0.0)
    hm = hnew * msk
    s_blk = jnp.sum(hm, axis=0, keepdims=True)
    ss_blk = jnp.sum(hnew * hm, axis=0, keepdims=True)
    blk_sums = jnp.concatenate([s_blk, ss_blk], axis=0)

    @pl.when(r == 0)
    def _():
        ns_ref[...] = blk_sums

    @pl.when(r > 0)
    def _():
        ns_ref[...] = ns_ref[...] + blk_sums


def _post_call(hraw, sums, g2, b2, aggc, h3c, dis, gb2, lw, lb2):
    return pl.pallas_call(
        _post_body,
        grid=(NBLK,),
        in_specs=[
            pl.BlockSpec((BLK, H), lambda r: (r, 0)),
            pl.BlockSpec((2, H), lambda r: (0, 0)),
            pl.BlockSpec((1, H), lambda r: (0, 0)),
            pl.BlockSpec((1, H), lambda r: (0, 0)),
            pl.BlockSpec((NCH, BLK, FC), lambda r: (0, r, 0)),
            pl.BlockSpec((NCH, BLK, FC), lambda r: (0, r, 0)),
            pl.BlockSpec((BLK, 1), lambda r: (r, 0)),
            pl.BlockSpec((1, H), lambda r: (0, 0)),
            pl.BlockSpec((H, H), lambda r: (0, 0)),
            pl.BlockSpec((1, H), lambda r: (0, 0)),
        ],
        out_specs=[
            pl.BlockSpec((BLK, H), lambda r: (r, 0)),
            pl.BlockSpec((2, H), lambda r: (0, 0)),
        ],
        out_shape=[
            jax.ShapeDtypeStruct((NR, H), jnp.float32),
            jax.ShapeDtypeStruct((2, H), jnp.float32),
        ],
    )(hraw, sums, g2, b2, aggc, h3c, dis, gb2, lw, lb2)


def _head_body(h_ref, sums_ref, g_ref, b_ref, w_ref, ob_ref, out_ref):
    a, cshift = _bn_affine(sums_ref, g_ref, b_ref)
    hn = h_ref[...] * a + cshift
    y = jnp.dot(hn, w_ref[...], preferred_element_type=jnp.float32)
    out_ref[...] = y + ob_ref[...]


def _head_call(hraw, sums, g2, b2, w, ob2):
    return pl.pallas_call(
        _head_body,
        grid=(NBLK,),
        in_specs=[
            pl.BlockSpec((BLK, H), lambda r: (r, 0)),
            pl.BlockSpec((2, H), lambda r: (0, 0)),
            pl.BlockSpec((1, H), lambda r: (0, 0)),
            pl.BlockSpec((1, H), lambda r: (0, 0)),
            pl.BlockSpec((H, OUTF), lambda r: (0, 0)),
            pl.BlockSpec((1, OUTF), lambda r: (0, 0)),
        ],
        out_specs=pl.BlockSpec((BLK, OUTF), lambda r: (r, 0)),
        out_shape=jax.ShapeDtypeStruct((NR, OUTF), jnp.float32),
    )(hraw, sums, g2, b2, w, ob2)


def kernel(x, params, edge_index):
    f32 = jnp.float32
    i32 = jnp.int32

    x_pad = jnp.zeros((NR, INF), f32).at[:N].set(x)
    pad_idx = jnp.full((EP - E,), PAD, dtype=i32)
    srcp = jnp.concatenate([edge_index[0].astype(i32), pad_idx])
    dstp = jnp.concatenate([edge_index[1].astype(i32), pad_idx])
    srcadj = (srcp[None, :]
              + (jnp.arange(NCH, dtype=i32) * NR)[:, None]
              ).reshape(NCH, 16, NQ, NBQ, 1, BA)
    dst16 = dstp.reshape(16, NQ, NBQ, 1, BA)
    dst32 = dstp.reshape(32, NB32, 1, B)
    ones_hbm = jnp.ones((B, DW), f32)
    z16 = jnp.zeros((RPT, DW), f32)
    zrows = jnp.zeros((RPT, FC), f32)

    def r2(v):
        return v.reshape(1, -1)

    degp = _deg_kernel(dst32, ones_hbm, z16)
    hraw, dis = _stem_call(x_pad, params['stem_W'], r2(params['stem_b']), degp)

    sums = jnp.concatenate(
        [jnp.zeros((1, H), f32), jnp.full((1, H), NF * (1.0 - EPS), f32)],
        axis=0)
    g2 = jnp.ones((1, H), f32)
    b2 = jnp.zeros((1, H), f32)

    for i in range(5):
        h3c = _pre_call(hraw, sums, g2, b2, params['gcn_W%d' % i], dis)
        aggc = _agg_kernel(h3c.reshape(NCH * NR, FC), srcadj, dst16, zrows)
        hraw, sums = _post_call(
            hraw, sums, g2, b2, aggc, h3c, dis,
            r2(params['gcn_b%d' % i]), params['lin_W%d' % i],
            r2(params['lin_b%d' % i]))
        g2 = r2(params['bn_g%d' % i])
        b2 = r2(params['bn_b%d' % i])

    y = _head_call(hraw, sums, g2, b2, params['out_W'], r2(params['out_b']))
    return y[:N]


# speedup vs baseline: 1.1085x; 1.0165x over previous
import functools

import jax
import jax.numpy as jnp
from jax import lax
from jax.experimental import pallas as pl
from jax.experimental.pallas import tpu as pltpu
from jax.experimental.pallas import tpu_sc as plsc

N = 10000
E = 160000
H = 1280
INF = 10
OUTF = 7

NR = 10240
PAD = N
EP = 163840
B = 128
NB32 = 40
BA = 128
NBQ = 40
NQ = 2
DW = 128
FC = 128
NCH = 10
RPT = NR // 16

BLK = 512
NBLK = NR // BLK
EPS = 1e-5
NF = float(N)

_mesh = plsc.VectorSubcoreMesh(core_axis_name="c", subcore_axis_name="s")


@functools.partial(
    pl.kernel,
    out_type=jax.ShapeDtypeStruct((2, NR, DW), jnp.float32),
    mesh=_mesh,
    scratch_types=[
        pltpu.VMEM((NB32, 1, B), jnp.int32),
        pltpu.VMEM((B, DW), jnp.float32),
        pltpu.VMEM_SHARED((NR, DW), jnp.float32),
    ],
)
def _deg_kernel(dst32, ones_hbm, z16, out, dst_v, ones_v, deg_sh):
    c = lax.axis_index("c")
    s = lax.axis_index("s")
    wid = s * 2 + c
    pltpu.sync_copy(ones_hbm, ones_v)
    pltpu.sync_copy(dst32.at[wid], dst_v)
    pltpu.sync_copy(z16, deg_sh.at[pl.ds(s * RPT, RPT)])
    plsc.subcore_barrier()

    def body(b, carry):
        pltpu.sync_copy(ones_v, deg_sh.at[dst_v.at[b, 0]], add=True)
        return carry

    lax.fori_loop(0, NB32, body, 0)
    plsc.subcore_barrier()
    pltpu.sync_copy(deg_sh.at[pl.ds(s * RPT, RPT)],
                    out.at[c, pl.ds(s * RPT, RPT)])


@functools.partial(
    pl.kernel,
    out_type=jax.ShapeDtypeStruct((NCH, NR, FC), jnp.float32),
    mesh=_mesh,
    scratch_types=[
        pltpu.VMEM((NBQ, 1, BA), jnp.int32),
        pltpu.VMEM((NBQ, 1, BA), jnp.int32),
        pltpu.VMEM((BA, FC), jnp.float32),
        pltpu.VMEM((BA, FC), jnp.float32),
        pltpu.VMEM_SHARED((NR, FC), jnp.float32),
        pltpu.SemaphoreType.DMA,
        pltpu.SemaphoreType.DMA,
        pltpu.SemaphoreType.DMA,
        pltpu.SemaphoreType.DMA,
    ],
)
def _agg_kernel(h3t, srcadj, dst16, zrows, out,
                src_v, dst_v, buf0, buf1,
                a_sh, gs0, gs1, ss0, ss1):
    c = lax.axis_index("c")
    s = lax.axis_index("s")
    bufs = (buf0, buf1)
    gsems = (gs0, gs1)
    ssems = (ss0, ss1)

    def issue_g(b, k):
        pltpu.async_copy(h3t.at[src_v.at[b, 0]], bufs[k], gsems[k])

    def drain_g(k):
        pltpu.make_async_copy(h3t.at[src_v.at[0, 0]], bufs[k],
                              gsems[k]).wait()

    def issue_s(b, k):
        pltpu.async_copy(bufs[k], a_sh.at[dst_v.at[b, 0]], ssems[k],
                         add=True)

    def drain_s(k):
        pltpu.make_async_copy(bufs[k], a_sh.at[dst_v.at[0, 0]],
                              ssems[k]).wait()

    def prime(p, q):
        ci_ = c * (NCH // 2) + p
        pltpu.sync_copy(srcadj.at[ci_, s, q], src_v)
        pltpu.sync_copy(dst16.at[s, q], dst_v)
        issue_g(0, 0)
        issue_g(1, 1)

    prime(0, 0)
    for p in range(NCH // 2):
        ci = c * (NCH // 2) + p
        pltpu.sync_copy(zrows, a_sh.at[pl.ds(s * RPT, RPT)])
        plsc.subcore_barrier()

        for q in range(NQ):
            def body(g, carry):
                for k in range(2):
                    b = g * 2 + k
                    drain_g(k)
                    issue_s(b, k)
                    drain_s(k)

                    @pl.when(b + 2 < NBQ)
                    def _():
                        issue_g(b + 2, k)

                return carry

            lax.fori_loop(0, NBQ // 2, body, 0)
            if q + 1 < NQ:
                prime(p, q + 1)
            elif p + 1 < NCH // 2:
                prime(p + 1, 0)

        plsc.subcore_barrier()
        pltpu.sync_copy(a_sh.at[pl.ds(s * RPT, RPT)],
                        out.at[ci, pl.ds(s * RPT, RPT)])
        plsc.subcore_barrier()


def _stem_body(x_ref, w_ref, b_ref, d_ref, h_ref, dis_ref):
    h = jnp.dot(x_ref[...], w_ref[...], preferred_element_type=jnp.float32)
    h_ref[...] = jnp.maximum(h + b_ref[...], 0.0)
    deg = d_ref[0, :, 0:1] + d_ref[1, :, 0:1] + 1.0
    dis_ref[...] = lax.rsqrt(deg)


def _stem_call(x_pad, w, b2, degp):
    return pl.pallas_call(
        _stem_body,
        grid=(NBLK,),
        in_specs=[
            pl.BlockSpec((BLK, INF), lambda r: (r, 0)),
            pl.BlockSpec((INF, H), lambda r: (0, 0)),
            pl.BlockSpec((1, H), lambda r: (0, 0)),
            pl.BlockSpec((2, BLK, DW), lambda r: (0, r, 0)),
        ],
        out_specs=[
            pl.BlockSpec((BLK, H), lambda r: (r, 0)),
            pl.BlockSpec((BLK, 1), lambda r: (r, 0)),
        ],
        out_shape=[
            jax.ShapeDtypeStruct((NR, H), jnp.float32),
            jax.ShapeDtypeStruct((NR, 1), jnp.float32),
        ],
    )(x_pad, w, b2, degp)


def _bn_affine(sums_ref, g_ref, b_ref):
    mean = sums_ref[0:1, :] * (1.0 / NF)
    var = sums_ref[1:2, :] * (1.0 / NF) - mean * mean
    a = g_ref[...] * lax.rsqrt(var + EPS)
    cshift = b_ref[...] - mean * a
    return a, cshift


def _pre_body(h_ref, sums_ref, g_ref, b_ref, w_ref, dis_ref, out_ref):
    a, cshift = _bn_affine(sums_ref, g_ref, b_ref)
    hn = h_ref[...] * a + cshift
    res = jnp.dot(hn, w_ref[...], preferred_element_type=jnp.float32)
    res = res * dis_ref[...]
    for k in range(NCH):
        out_ref[k] = res[:, k * FC:(k + 1) * FC]


def _pre_call(hraw, sums, g2, b2, w, dis):
    return pl.pallas_call(
        _pre_body,
        grid=(NBLK,),
        in_specs=[
            pl.BlockSpec((BLK, H), lambda r: (r, 0)),
            pl.BlockSpec((2, H), lambda r: (0, 0)),
            pl.BlockSpec((1, H), lambda r: (0, 0)),
            pl.BlockSpec((1, H), lambda r: (0, 0)),
            pl.BlockSpec((H, H), lambda r: (0, 0)),
            pl.BlockSpec((BLK, 1), lambda r: (r, 0)),
        ],
        out_specs=pl.BlockSpec((NCH, BLK, FC), lambda r: (0, r, 0)),
        out_shape=jax.ShapeDtypeStruct((NCH, NR, FC), jnp.float32),
    )(hraw, sums, g2, b2, w, dis)


def _post_body(h_ref, sums_ref, g_ref, b_ref, agg_ref, h3_ref, dis_ref,
               gb_ref, lw_ref, lb_ref, out_ref, ns_ref):
    r = pl.program_id(0)
    a, cshift = _bn_affine(sums_ref, g_ref, b_ref)
    hprev = h_ref[...] * a + cshift
    agg = jnp.concatenate([agg_ref[k] + h3_ref[k] for k in range(NCH)],
                          axis=1)
    t = jnp.maximum(agg * dis_ref[...] + gb_ref[...], 0.0)
    u = jnp.dot(t, lw_ref[...], preferred_element_type=jnp.float32)
    u = jnp.maximum(u + lb_ref[...], 0.0)
    hnew = hprev + u
    out_ref[...] = hnew
    rows = r * BLK + lax.broadcasted_iota(jnp.int32, (BLK, 1), 0)
    msk = jnp.where(rows < N, 1.0, 0.0)
    hm = hnew * msk
    s_blk = jnp.sum(hm, axis=0, keepdims=True)
    ss_blk = jnp.sum(hnew * hm, axis=0, keepdims=True)
    blk_sums = jnp.concatenate([s_blk, ss_blk], axis=0)

    @pl.when(r == 0)
    def _():
        ns_ref[...] = blk_sums

    @pl.when(r > 0)
    def _():
        ns_ref[...] = ns_ref[...] + blk_sums


def _post_call(hraw, sums, g2, b2, aggc, h3c, dis, gb2, lw, lb2):
    return pl.pallas_call(
        _post_body,
        grid=(NBLK,),
        in_specs=[
            pl.BlockSpec((BLK, H), lambda r: (r, 0)),
            pl.BlockSpec((2, H), lambda r: (0, 0)),
            pl.BlockSpec((1, H), lambda r: (0, 0)),
            pl.BlockSpec((1, H), lambda r: (0, 0)),
            pl.BlockSpec((NCH, BLK, FC), lambda r: (0, r, 0)),
            pl.BlockSpec((NCH, BLK, FC), lambda r: (0, r, 0)),
            pl.BlockSpec((BLK, 1), lambda r: (r, 0)),
            pl.BlockSpec((1, H), lambda r: (0, 0)),
            pl.BlockSpec((H, H), lambda r: (0, 0)),
            pl.BlockSpec((1, H), lambda r: (0, 0)),
        ],
        out_specs=[
            pl.BlockSpec((BLK, H), lambda r: (r, 0)),
            pl.BlockSpec((2, H), lambda r: (0, 0)),
        ],
        out_shape=[
            jax.ShapeDtypeStruct((NR, H), jnp.float32),
            jax.ShapeDtypeStruct((2, H), jnp.float32),
        ],
    )(hraw, sums, g2, b2, aggc, h3c, dis, gb2, lw, lb2)


def _head_body(h_ref, sums_ref, g_ref, b_ref, w_ref, ob_ref, out_ref):
    a, cshift = _bn_affine(sums_ref, g_ref, b_ref)
    hn = h_ref[...] * a + cshift
    y = jnp.dot(hn, w_ref[...], preferred_element_type=jnp.float32)
    out_ref[...] = y + ob_ref[...]


def _head_call(hraw, sums, g2, b2, w, ob2):
    return pl.pallas_call(
        _head_body,
        grid=(NBLK,),
        in_specs=[
            pl.BlockSpec((BLK, H), lambda r: (r, 0)),
            pl.BlockSpec((2, H), lambda r: (0, 0)),
            pl.BlockSpec((1, H), lambda r: (0, 0)),
            pl.BlockSpec((1, H), lambda r: (0, 0)),
            pl.BlockSpec((H, OUTF), lambda r: (0, 0)),
            pl.BlockSpec((1, OUTF), lambda r: (0, 0)),
        ],
        out_specs=pl.BlockSpec((BLK, OUTF), lambda r: (r, 0)),
        out_shape=jax.ShapeDtypeStruct((NR, OUTF), jnp.float32),
    )(hraw, sums, g2, b2, w, ob2)


def kernel(x, params, edge_index):
    f32 = jnp.float32
    i32 = jnp.int32

    x_pad = jnp.zeros((NR, INF), f32).at[:N].set(x)
    pad_idx = jnp.full((EP - E,), PAD, dtype=i32)
    srcp = jnp.concatenate([edge_index[0].astype(i32), pad_idx])
    dstp = jnp.concatenate([edge_index[1].astype(i32), pad_idx])
    srcadj = (srcp[None, :]
              + (jnp.arange(NCH, dtype=i32) * NR)[:, None]
              ).reshape(NCH, 16, NQ, NBQ, 1, BA)
    dst16 = dstp.reshape(16, NQ, NBQ, 1, BA)
    dst32 = dstp.reshape(32, NB32, 1, B)
    ones_hbm = jnp.ones((B, DW), f32)
    z16 = jnp.zeros((RPT, DW), f32)
    zrows = jnp.zeros((RPT, FC), f32)

    def r2(v):
        return v.reshape(1, -1)

    degp = _deg_kernel(dst32, ones_hbm, z16)
    hraw, dis = _stem_call(x_pad, params['stem_W'], r2(params['stem_b']), degp)

    sums = jnp.concatenate(
        [jnp.zeros((1, H), f32), jnp.full((1, H), NF * (1.0 - EPS), f32)],
        axis=0)
    g2 = jnp.ones((1, H), f32)
    b2 = jnp.zeros((1, H), f32)

    for i in range(5):
        h3c = _pre_call(hraw, sums, g2, b2, params['gcn_W%d' % i], dis)
        aggc = _agg_kernel(h3c.reshape(NCH * NR, FC), srcadj, dst16, zrows)
        hraw, sums = _post_call(
            hraw, sums, g2, b2, aggc, h3c, dis,
            r2(params['gcn_b%d' % i]), params['lin_W%d' % i],
            r2(params['lin_b%d' % i]))
        g2 = r2(params['bn_g%d' % i])
        b2 = r2(params['bn_b%d' % i])

    y = _head_call(hraw, sums, g2, b2, params['out_W'], r2(params['out_b']))
    return y[:N]
